# Initial kernel scaffold; baseline (speedup 1.0000x reference)
import functools

import jax
import jax.numpy as jnp
from jax import lax
from jax.experimental import pallas as pl
from jax.experimental.pallas import tpu as pltpu
from jax.experimental.pallas import tpu_sc as plsc

N = 10000
E = 320000
D = 128
H = 128
C = 40
CP = 48

NC = 2
NS = 16
K = 80
EDGES_PER_SUB = E // (NC * NS)
CHUNKS = EDGES_PER_SUB // K

_f32 = jnp.float32


def _mesh():
    return plsc.VectorSubcoreMesh(core_axis_name="c", subcore_axis_name="s")


def _make_deg_kernel():
    @functools.partial(
        pl.kernel,
        out_type=jax.ShapeDtypeStruct((NC * N,), _f32),
        mesh=_mesh(),
        scratch_types=[
            pltpu.VMEM((K,), jnp.int32),
            pltpu.VMEM((K,), _f32),
            pltpu.VMEM_SHARED((N,), _f32),
        ],
    )
    def deg_kernel(dst_hbm, zeros_hbm, out_hbm, didx, ones, table):
        cid = lax.axis_index("c")
        sid = lax.axis_index("s")
        for j in range(K // 16):
            ones[pl.ds(16 * j, 16)] = jnp.full((16,), 1.0, _f32)

        @pl.when(sid < 5)
        def _():
            pltpu.sync_copy(zeros_hbm.at[pl.ds(sid * 2000, 2000)],
                            table.at[pl.ds(sid * 2000, 2000)])

        plsc.subcore_barrier()

        base = (cid * NS + sid) * EDGES_PER_SUB

        @pl.loop(0, CHUNKS)
        def _(i):
            pltpu.sync_copy(dst_hbm.at[pl.ds(base + i * K, K)], didx)
            pltpu.sync_copy(ones, table.at[didx], add=True)

        plsc.subcore_barrier()

        @pl.when(sid < 5)
        def _():
            pltpu.sync_copy(table.at[pl.ds(sid * 2000, 2000)],
                            out_hbm.at[pl.ds(cid * N + sid * 2000, 2000)])

    return deg_kernel


def _make_scatter_kernel(F):
    @functools.partial(
        pl.kernel,
        out_type=jax.ShapeDtypeStruct((NC * N, F), _f32),
        mesh=_mesh(),
        scratch_types=[
            pltpu.VMEM((K,), jnp.int32),
            pltpu.VMEM((K,), jnp.int32),
            pltpu.VMEM((K, F), _f32),
            pltpu.VMEM_SHARED((N, F), _f32),
            pltpu.SemaphoreType.DMA,
        ],
    )
    def scatter_kernel(hs_hbm, src_hbm, dst_hbm, zeros_hbm, out_hbm,
                       sidx, didx, rbuf, acc, sem):
        cid = lax.axis_index("c")
        sid = lax.axis_index("s")

        @pl.when(sid < 10)
        def _():
            pltpu.sync_copy(zeros_hbm.at[pl.ds(sid * 1000, 1000)],
                            acc.at[pl.ds(sid * 1000, 1000)])

        plsc.subcore_barrier()

        base = (cid * NS + sid) * EDGES_PER_SUB

        @pl.loop(0, CHUNKS)
        def _(i):
            off = base + i * K
            pltpu.sync_copy(src_hbm.at[pl.ds(off, K)], sidx)
            pltpu.sync_copy(dst_hbm.at[pl.ds(off, K)], didx)
            pltpu.async_copy(hs_hbm.at[sidx], rbuf, sem).wait()
            pltpu.sync_copy(rbuf, acc.at[didx], add=True)

        plsc.subcore_barrier()

        @pl.when(sid < 10)
        def _():
            pltpu.sync_copy(acc.at[pl.ds(sid * 1000, 1000)],
                            out_hbm.at[pl.ds(cid * N + sid * 1000, 1000)])

    return scatter_kernel


_deg_call = _make_deg_kernel()
_scatter128 = _make_scatter_kernel(D)
_scatter48 = _make_scatter_kernel(CP)


def _dinv_of(degp):
    return lax.rsqrt(degp[0] + degp[1] + 1.0)


def _mm1_body(x_ref, w_ref, o_ref):
    o_ref[...] = jnp.dot(x_ref[...], w_ref[...],
                         preferred_element_type=_f32,
                         precision=lax.Precision.HIGHEST)


def _scale_body(degp_ref, h_ref, o_ref):
    dinv = _dinv_of(degp_ref[...])
    o_ref[...] = h_ref[...] * dinv[:, None]


def _layer2_body(degp_ref, agg_ref, hs1_ref, b1_ref, w2_ref, o_ref):
    dinv = _dinv_of(degp_ref[...])
    a = agg_ref[:N] + agg_ref[N:] + hs1_ref[...]
    out1 = jnp.maximum(dinv[:, None] * a + b1_ref[...][None, :], 0.0)
    h2 = jnp.dot(out1, w2_ref[...],
                 preferred_element_type=_f32,
                 precision=lax.Precision.HIGHEST)
    o_ref[...] = h2 * dinv[:, None]


def _final_body(degp_ref, agg_ref, h2s_ref, b2_ref, o_ref):
    dinv = _dinv_of(degp_ref[...])
    a = agg_ref[:N] + agg_ref[N:] + h2s_ref[...]
    z = dinv[:, None] * a[:, :C] + b2_ref[...][None, :]
    z = jnp.maximum(z, 0.0)
    m = jnp.max(z, axis=1, keepdims=True)
    e = z - m
    lse = jnp.log(jnp.sum(jnp.exp(e), axis=1, keepdims=True))
    o_ref[...] = e - lse


def _tc(body, out_shape):
    return pl.pallas_call(body, out_shape=jax.ShapeDtypeStruct(out_shape, _f32))


def kernel(x, edge_index, W1, b1, W2, b2):
    src = edge_index[0].astype(jnp.int32)
    dst = edge_index[1].astype(jnp.int32)
    W2p = jnp.pad(W2, ((0, 0), (0, CP - C)))

    zeros1 = jnp.zeros((N,), _f32)
    zerosD = jnp.zeros((N, D), _f32)
    zerosCP = jnp.zeros((N, CP), _f32)

    degp = _deg_call(dst, zeros1).reshape(NC, N)

    h1 = _tc(_mm1_body, (N, D))(x, W1)
    hs1 = _tc(_scale_body, (N, D))(degp, h1)

    agg1 = _scatter128(hs1, src, dst, zerosD)

    h2s = _tc(_layer2_body, (N, CP))(degp, agg1, hs1, b1, W2p)

    agg2 = _scatter48(h2s, src, dst, zerosCP)

    return _tc(_final_body, (N, C))(degp, agg2, h2s, b2)


# speedup vs baseline: 13.4636x; 13.4636x over previous
import functools

import jax
import jax.numpy as jnp
from jax import lax
from jax.experimental import pallas as pl
from jax.experimental.pallas import tpu as pltpu
from jax.experimental.pallas import tpu_sc as plsc

N = 10000
E = 320000
D = 128
H = 128
C = 40
CP = 48

NC = 2
NS = 16
K = 80
EDGES_PER_SUB = E // (NC * NS)
CHUNKS = EDGES_PER_SUB // K

_f32 = jnp.float32


def _mesh():
    return plsc.VectorSubcoreMesh(core_axis_name="c", subcore_axis_name="s")


def _make_deg_kernel():
    NU = N // K
    NT = -(-NU // NS)

    @functools.partial(
        pl.kernel,
        out_type=jax.ShapeDtypeStruct((NC * N,), _f32),
        mesh=_mesh(),
        scratch_types=[
            pltpu.VMEM((K,), jnp.int32),
            pltpu.VMEM((K,), _f32),
            pltpu.VMEM((K,), _f32),
            pltpu.VMEM_SHARED((N,), _f32),
        ],
    )
    def deg_kernel(dst_hbm, out_hbm, didx, ones, vbuf, table):
        cid = lax.axis_index("c")
        sid = lax.axis_index("s")
        for j in range(K // 16):
            ones[pl.ds(16 * j, 16)] = jnp.full((16,), 1.0, _f32)
            vbuf[pl.ds(16 * j, 16)] = jnp.zeros((16,), _f32)

        @pl.loop(0, NT)
        def _(t):
            u = t * NS + sid

            @pl.when(u < NU)
            def _():
                pltpu.sync_copy(vbuf, table.at[pl.ds(u * K, K)])

        plsc.subcore_barrier()

        base = (cid * NS + sid) * EDGES_PER_SUB

        @pl.loop(0, CHUNKS)
        def _(i):
            pltpu.sync_copy(dst_hbm.at[pl.ds(base + i * K, K)], didx)
            pltpu.sync_copy(ones, table.at[didx], add=True)

        plsc.subcore_barrier()

        @pl.loop(0, NT)
        def _(t):
            u = t * NS + sid

            @pl.when(u < NU)
            def _():
                pltpu.sync_copy(table.at[pl.ds(u * K, K)], vbuf)
                pltpu.sync_copy(vbuf, out_hbm.at[pl.ds(cid * N + u * K, K)])

    return deg_kernel


def _make_scatter_kernel(F):
    NU = N // K
    NT = -(-NU // NS)

    @functools.partial(
        pl.kernel,
        out_type=jax.ShapeDtypeStruct((NC * N, F), _f32),
        mesh=_mesh(),
        scratch_types=[
            pltpu.VMEM((K,), jnp.int32),
            pltpu.VMEM((K,), jnp.int32),
            pltpu.VMEM((K, F), _f32),
            pltpu.VMEM_SHARED((N, F), _f32),
            pltpu.SemaphoreType.DMA,
        ],
    )
    def scatter_kernel(hs_hbm, src_hbm, dst_hbm, out_hbm,
                       sidx, didx, rbuf, acc, sem):
        cid = lax.axis_index("c")
        sid = lax.axis_index("s")

        @pl.loop(0, K)
        def _(r):
            for j in range(F // 16):
                rbuf[r, pl.ds(16 * j, 16)] = jnp.zeros((16,), _f32)

        @pl.loop(0, NT)
        def _(t):
            u = t * NS + sid

            @pl.when(u < NU)
            def _():
                pltpu.sync_copy(rbuf, acc.at[pl.ds(u * K, K)])

        plsc.subcore_barrier()

        base = (cid * NS + sid) * EDGES_PER_SUB

        @pl.loop(0, CHUNKS)
        def _(i):
            off = base + i * K
            pltpu.sync_copy(src_hbm.at[pl.ds(off, K)], sidx)
            pltpu.sync_copy(dst_hbm.at[pl.ds(off, K)], didx)
            pltpu.async_copy(hs_hbm.at[sidx], rbuf, sem).wait()
            pltpu.sync_copy(rbuf, acc.at[didx], add=True)

        plsc.subcore_barrier()

        @pl.loop(0, NT)
        def _(t):
            u = t * NS + sid

            @pl.when(u < NU)
            def _():
                pltpu.sync_copy(acc.at[pl.ds(u * K, K)], rbuf)
                pltpu.sync_copy(rbuf, out_hbm.at[pl.ds(cid * N + u * K, K)])

    return scatter_kernel


_deg_call = _make_deg_kernel()
_scatter128 = _make_scatter_kernel(D)


def _dinv_of(degp):
    return lax.rsqrt(degp[0] + degp[1] + 1.0)


def _mm1_body(x_ref, w_ref, o_ref):
    o_ref[...] = jnp.dot(x_ref[...], w_ref[...],
                         preferred_element_type=_f32,
                         precision=lax.Precision.HIGHEST)


def _scale_body(degp_ref, h_ref, o_ref):
    dinv = _dinv_of(degp_ref[...])
    o_ref[...] = h_ref[...] * dinv[:, None]


def _layer2_body(degp_ref, agg_ref, hs1_ref, b1_ref, o_ref):
    dinv = _dinv_of(degp_ref[...])
    a = agg_ref[:N] + agg_ref[N:] + hs1_ref[...]
    out1 = jnp.maximum(dinv[:, None] * a + b1_ref[...][None, :], 0.0)
    o_ref[...] = out1 * dinv[:, None]


def _final_body(degp_ref, agg_ref, os1_ref, b2_ref, w2_ref, o_ref):
    dinv = _dinv_of(degp_ref[...])
    a = agg_ref[:N] + agg_ref[N:] + os1_ref[...]
    t = jnp.dot(a, w2_ref[...],
                preferred_element_type=_f32,
                precision=lax.Precision.HIGHEST)
    z = dinv[:, None] * t + b2_ref[...][None, :]
    z = jnp.maximum(z, 0.0)
    m = jnp.max(z, axis=1, keepdims=True)
    e = z - m
    lse = jnp.log(jnp.sum(jnp.exp(e), axis=1, keepdims=True))
    o_ref[...] = e - lse


def _tc(body, out_shape):
    return pl.pallas_call(body, out_shape=jax.ShapeDtypeStruct(out_shape, _f32))


def kernel(x, edge_index, W1, b1, W2, b2):
    src = edge_index[0].astype(jnp.int32)
    dst = edge_index[1].astype(jnp.int32)

    degp = _deg_call(dst).reshape(NC, N)

    h1 = _tc(_mm1_body, (N, D))(x, W1)
    hs1 = _tc(_scale_body, (N, D))(degp, h1)

    agg1 = _scatter128(hs1, src, dst)

    os1 = _tc(_layer2_body, (N, D))(degp, agg1, hs1, b1)

    agg2 = _scatter128(os1, src, dst)

    return _tc(_final_body, (N, C))(degp, agg2, os1, b2, W2)


# speedup vs baseline: 36.6683x; 2.7235x over previous
import functools

import jax
import jax.numpy as jnp
from jax import lax
from jax.experimental import pallas as pl
from jax.experimental.pallas import tpu as pltpu
from jax.experimental.pallas import tpu_sc as plsc

N = 10000
E = 320000
D = 128
H = 128
C = 40
CP = 48

NC = 2
NS = 16
K = 80
EDGES_PER_SUB = E // (NC * NS)
CHUNKS = EDGES_PER_SUB // K
NB = 4
NBS = 3

_f32 = jnp.float32


def _mesh():
    return plsc.VectorSubcoreMesh(core_axis_name="c", subcore_axis_name="s")


def _make_deg_kernel():
    NU = N // K
    NT = -(-NU // NS)

    @functools.partial(
        pl.kernel,
        out_type=jax.ShapeDtypeStruct((NC * N,), _f32),
        mesh=_mesh(),
        scratch_types=(
            [pltpu.VMEM((K,), jnp.int32) for _ in range(NB)]
            + [pltpu.VMEM((K,), _f32),
               pltpu.VMEM((K,), _f32),
               pltpu.VMEM_SHARED((N,), _f32)]
            + [pltpu.SemaphoreType.DMA for _ in range(2 * NB)]
        ),
    )
    def deg_kernel(dst_hbm, out_hbm, *refs):
        dbuf = refs[0:NB]
        ones, vbuf, table = refs[NB:NB + 3]
        isem = refs[NB + 3:2 * NB + 3]
        ssem = refs[2 * NB + 3:3 * NB + 3]
        cid = lax.axis_index("c")
        sid = lax.axis_index("s")
        base = (cid * NS + sid) * EDGES_PER_SUB
        for j in range(K // 16):
            ones[pl.ds(16 * j, 16)] = jnp.full((16,), 1.0, _f32)
            vbuf[pl.ds(16 * j, 16)] = jnp.zeros((16,), _f32)

        for b in range(NB):
            pltpu.async_copy(dst_hbm.at[pl.ds(base + b * K, K)], dbuf[b],
                             isem[b])

        @pl.loop(0, NT)
        def _(t):
            u = t * NS + sid

            @pl.when(u < NU)
            def _():
                pltpu.sync_copy(vbuf, table.at[pl.ds(u * K, K)])

        plsc.subcore_barrier()

        @pl.loop(0, CHUNKS, step=NB)
        def _(i):
            for b in range(NB):
                c = i + b

                @pl.when(c < CHUNKS)
                def _():
                    pltpu.make_async_copy(dst_hbm.at[pl.ds(base, K)],
                                          dbuf[b], isem[b]).wait()
                    pltpu.async_copy(ones, table.at[dbuf[b]], ssem[b],
                                     add=True)

                    @pl.when(c + NB < CHUNKS)
                    def _():
                        pltpu.make_async_copy(ones, table.at[dbuf[b]],
                                              ssem[b]).wait()
                        pltpu.async_copy(
                            dst_hbm.at[pl.ds(base + (c + NB) * K, K)],
                            dbuf[b], isem[b])

        for b in range(NB):
            pltpu.make_async_copy(ones, table.at[dbuf[b]], ssem[b]).wait()

        plsc.subcore_barrier()

        @pl.loop(0, NT)
        def _(t):
            u = t * NS + sid

            @pl.when(u < NU)
            def _():
                pltpu.sync_copy(table.at[pl.ds(u * K, K)], vbuf)
                pltpu.sync_copy(vbuf, out_hbm.at[pl.ds(cid * N + u * K, K)])

    return deg_kernel


def _make_scatter_kernel(F):
    NU = N // K
    NT = -(-NU // NS)

    @functools.partial(
        pl.kernel,
        out_type=jax.ShapeDtypeStruct((NC * N, F), _f32),
        mesh=_mesh(),
        scratch_types=(
            [pltpu.VMEM((EDGES_PER_SUB,), jnp.int32)]
            + [pltpu.VMEM((K,), jnp.int32) for _ in range(NBS)]
            + [pltpu.VMEM((K, F), _f32) for _ in range(NBS)]
            + [pltpu.VMEM_SHARED((N, F), _f32)]
            + [pltpu.SemaphoreType.DMA for _ in range(3 * NBS)]
        ),
    )
    def scatter_kernel(hs_hbm, src_hbm, dst_hbm, out_hbm, *refs):
        sidx_v = refs[0]
        dbuf = refs[1:1 + NBS]
        rbuf = refs[1 + NBS:1 + 2 * NBS]
        acc = refs[1 + 2 * NBS]
        isem = refs[2 + 2 * NBS:2 + 3 * NBS]
        gsem = refs[2 + 3 * NBS:2 + 4 * NBS]
        ssem = refs[2 + 4 * NBS:2 + 5 * NBS]
        cid = lax.axis_index("c")
        sid = lax.axis_index("s")
        base = (cid * NS + sid) * EDGES_PER_SUB

        pltpu.sync_copy(src_hbm.at[pl.ds(base, EDGES_PER_SUB)], sidx_v)

        @pl.loop(0, K)
        def _(r):
            for j in range(F // 16):
                rbuf[0][r, pl.ds(16 * j, 16)] = jnp.zeros((16,), _f32)

        @pl.loop(0, NT)
        def _(t):
            u = t * NS + sid

            @pl.when(u < NU)
            def _():
                pltpu.sync_copy(rbuf[0], acc.at[pl.ds(u * K, K)])

        for b in range(NBS):
            pltpu.async_copy(dst_hbm.at[pl.ds(base + b * K, K)], dbuf[b],
                             isem[b])
            pltpu.async_copy(hs_hbm.at[sidx_v.at[pl.ds(b * K, K)]], rbuf[b],
                             gsem[b])

        plsc.subcore_barrier()

        @pl.loop(0, CHUNKS, step=NBS)
        def _(i):
            for b in range(NBS):
                c = i + b

                @pl.when(c < CHUNKS)
                def _():
                    pltpu.make_async_copy(dst_hbm.at[pl.ds(base, K)],
                                          dbuf[b], isem[b]).wait()
                    pltpu.make_async_copy(
                        hs_hbm.at[sidx_v.at[pl.ds(0, K)]], rbuf[b],
                        gsem[b]).wait()
                    pltpu.async_copy(rbuf[b], acc.at[dbuf[b]], ssem[b],
                                     add=True)

                    @pl.when(c + NBS < CHUNKS)
                    def _():
                        nxt = c + NBS
                        pltpu.make_async_copy(rbuf[b], acc.at[dbuf[b]],
                                              ssem[b]).wait()
                        pltpu.async_copy(
                            dst_hbm.at[pl.ds(base + nxt * K, K)], dbuf[b],
                            isem[b])
                        pltpu.async_copy(
                            hs_hbm.at[sidx_v.at[pl.ds(nxt * K, K)]], rbuf[b],
                            gsem[b])

        for b in range(NBS):
            pltpu.make_async_copy(rbuf[b], acc.at[dbuf[b]], ssem[b]).wait()

        plsc.subcore_barrier()

        @pl.loop(0, NT)
        def _(t):
            u = t * NS + sid

            @pl.when(u < NU)
            def _():
                pltpu.sync_copy(acc.at[pl.ds(u * K, K)], rbuf[0])
                pltpu.sync_copy(rbuf[0],
                                out_hbm.at[pl.ds(cid * N + u * K, K)])

    return scatter_kernel


_deg_call = _make_deg_kernel()
_scatter128 = _make_scatter_kernel(D)


def _dinv_of(degp):
    return lax.rsqrt(degp[0] + degp[1] + 1.0)


def _mm1_body(x_ref, w_ref, o_ref):
    o_ref[...] = jnp.dot(x_ref[...], w_ref[...],
                         preferred_element_type=_f32,
                         precision=lax.Precision.HIGHEST)


def _scale_body(degp_ref, h_ref, o_ref):
    dinv = _dinv_of(degp_ref[...])
    o_ref[...] = h_ref[...] * dinv[:, None]


def _layer2_body(degp_ref, agg_ref, hs1_ref, b1_ref, o_ref):
    dinv = _dinv_of(degp_ref[...])
    a = agg_ref[:N] + agg_ref[N:] + hs1_ref[...]
    out1 = jnp.maximum(dinv[:, None] * a + b1_ref[...][None, :], 0.0)
    o_ref[...] = out1 * dinv[:, None]


def _final_body(degp_ref, agg_ref, os1_ref, b2_ref, w2_ref, o_ref):
    dinv = _dinv_of(degp_ref[...])
    a = agg_ref[:N] + agg_ref[N:] + os1_ref[...]
    t = jnp.dot(a, w2_ref[...],
                preferred_element_type=_f32,
                precision=lax.Precision.HIGHEST)
    z = dinv[:, None] * t + b2_ref[...][None, :]
    z = jnp.maximum(z, 0.0)
    m = jnp.max(z, axis=1, keepdims=True)
    e = z - m
    lse = jnp.log(jnp.sum(jnp.exp(e), axis=1, keepdims=True))
    o_ref[...] = e - lse


def _tc(body, out_shape):
    return pl.pallas_call(body, out_shape=jax.ShapeDtypeStruct(out_shape, _f32))


def kernel(x, edge_index, W1, b1, W2, b2):
    src = edge_index[0].astype(jnp.int32)
    dst = edge_index[1].astype(jnp.int32)

    degp = _deg_call(dst).reshape(NC, N)

    h1 = _tc(_mm1_body, (N, D))(x, W1)
    hs1 = _tc(_scale_body, (N, D))(degp, h1)

    agg1 = _scatter128(hs1, src, dst)

    os1 = _tc(_layer2_body, (N, D))(degp, agg1, hs1, b1)

    agg2 = _scatter128(os1, src, dst)

    return _tc(_final_body, (N, C))(degp, agg2, os1, b2, W2)


# speedup vs baseline: 36.7284x; 1.0016x over previous
import functools

import jax
import jax.numpy as jnp
from jax import lax
from jax.experimental import pallas as pl
from jax.experimental.pallas import tpu as pltpu
from jax.experimental.pallas import tpu_sc as plsc

N = 10000
E = 320000
D = 128
H = 128
C = 40
CP = 48

NC = 2
NS = 16
K = 80
EDGES_PER_SUB = E // (NC * NS)
CHUNKS = EDGES_PER_SUB // K
NB = 4
NBS = 3

_f32 = jnp.float32


def _mesh():
    return plsc.VectorSubcoreMesh(core_axis_name="c", subcore_axis_name="s")


def _make_deg_kernel():
    NU = N // K
    NT = -(-NU // NS)

    @functools.partial(
        pl.kernel,
        out_type=jax.ShapeDtypeStruct((NC * N,), _f32),
        mesh=_mesh(),
        scratch_types=(
            [pltpu.VMEM((K,), jnp.int32) for _ in range(NB)]
            + [pltpu.VMEM((K,), _f32),
               pltpu.VMEM((K,), _f32),
               pltpu.VMEM_SHARED((N,), _f32)]
            + [pltpu.SemaphoreType.DMA for _ in range(2 * NB)]
        ),
    )
    def deg_kernel(dst_hbm, out_hbm, *refs):
        dbuf = refs[0:NB]
        ones, vbuf, table = refs[NB:NB + 3]
        isem = refs[NB + 3:2 * NB + 3]
        ssem = refs[2 * NB + 3:3 * NB + 3]
        cid = lax.axis_index("c")
        sid = lax.axis_index("s")
        base = (cid * NS + sid) * EDGES_PER_SUB
        for j in range(K // 16):
            ones[pl.ds(16 * j, 16)] = jnp.full((16,), 1.0, _f32)
            vbuf[pl.ds(16 * j, 16)] = jnp.zeros((16,), _f32)

        for b in range(NB):
            pltpu.async_copy(dst_hbm.at[pl.ds(base + b * K, K)], dbuf[b],
                             isem[b])

        @pl.loop(0, NT)
        def _(t):
            u = t * NS + sid

            @pl.when(u < NU)
            def _():
                pltpu.async_copy(vbuf, table.at[pl.ds(u * K, K)], ssem[0])

        @pl.loop(0, NT)
        def _(t):
            u = t * NS + sid

            @pl.when(u < NU)
            def _():
                pltpu.make_async_copy(vbuf, table.at[pl.ds(u * K, K)],
                                      ssem[0]).wait()

        plsc.subcore_barrier()

        @pl.loop(0, CHUNKS, step=NB)
        def _(i):
            for b in range(NB):
                c = i + b

                @pl.when(c < CHUNKS)
                def _():
                    pltpu.make_async_copy(dst_hbm.at[pl.ds(base, K)],
                                          dbuf[b], isem[b]).wait()
                    pltpu.async_copy(ones, table.at[dbuf[b]], ssem[b],
                                     add=True)

                    @pl.when(c + NB < CHUNKS)
                    def _():
                        pltpu.make_async_copy(ones, table.at[dbuf[b]],
                                              ssem[b]).wait()
                        pltpu.async_copy(
                            dst_hbm.at[pl.ds(base + (c + NB) * K, K)],
                            dbuf[b], isem[b])

        for b in range(NB):
            pltpu.make_async_copy(ones, table.at[dbuf[b]], ssem[b]).wait()

        plsc.subcore_barrier()

        @pl.loop(0, NT)
        def _(t):
            u = t * NS + sid

            @pl.when(u < NU)
            def _():
                pltpu.sync_copy(table.at[pl.ds(u * K, K)], vbuf)
                pltpu.sync_copy(vbuf, out_hbm.at[pl.ds(cid * N + u * K, K)])

    return deg_kernel


def _make_scatter_kernel(F):
    NU = N // K
    NT = -(-NU // NS)

    @functools.partial(
        pl.kernel,
        out_type=jax.ShapeDtypeStruct((NC * N, F), _f32),
        mesh=_mesh(),
        scratch_types=(
            [pltpu.VMEM((EDGES_PER_SUB,), jnp.int32)]
            + [pltpu.VMEM((K,), jnp.int32) for _ in range(NBS)]
            + [pltpu.VMEM((K, F), _f32) for _ in range(NBS)]
            + [pltpu.VMEM_SHARED((N, F), _f32)]
            + [pltpu.SemaphoreType.DMA for _ in range(3 * NBS)]
        ),
    )
    def scatter_kernel(hs_hbm, src_hbm, dst_hbm, out_hbm, *refs):
        sidx_v = refs[0]
        dbuf = refs[1:1 + NBS]
        rbuf = refs[1 + NBS:1 + 2 * NBS]
        acc = refs[1 + 2 * NBS]
        isem = refs[2 + 2 * NBS:2 + 3 * NBS]
        gsem = refs[2 + 3 * NBS:2 + 4 * NBS]
        ssem = refs[2 + 4 * NBS:2 + 5 * NBS]
        cid = lax.axis_index("c")
        sid = lax.axis_index("s")
        base = (cid * NS + sid) * EDGES_PER_SUB

        pltpu.sync_copy(src_hbm.at[pl.ds(base, EDGES_PER_SUB)], sidx_v)

        @pl.loop(0, K)
        def _(r):
            for j in range(F // 16):
                rbuf[0][r, pl.ds(16 * j, 16)] = jnp.zeros((16,), _f32)

        @pl.loop(0, NT)
        def _(t):
            u = t * NS + sid

            @pl.when(u < NU)
            def _():
                pltpu.async_copy(rbuf[0], acc.at[pl.ds(u * K, K)], gsem[0])

        @pl.loop(0, NT)
        def _(t):
            u = t * NS + sid

            @pl.when(u < NU)
            def _():
                pltpu.make_async_copy(rbuf[0], acc.at[pl.ds(u * K, K)],
                                      gsem[0]).wait()

        for b in range(NBS):
            pltpu.async_copy(dst_hbm.at[pl.ds(base + b * K, K)], dbuf[b],
                             isem[b])
            pltpu.async_copy(hs_hbm.at[sidx_v.at[pl.ds(b * K, K)]], rbuf[b],
                             gsem[b])

        plsc.subcore_barrier()

        @pl.loop(0, CHUNKS, step=NBS)
        def _(i):
            for b in range(NBS):
                c = i + b

                @pl.when(c < CHUNKS)
                def _():
                    pltpu.make_async_copy(dst_hbm.at[pl.ds(base, K)],
                                          dbuf[b], isem[b]).wait()
                    pltpu.make_async_copy(
                        hs_hbm.at[sidx_v.at[pl.ds(0, K)]], rbuf[b],
                        gsem[b]).wait()
                    pltpu.async_copy(rbuf[b], acc.at[dbuf[b]], ssem[b],
                                     add=True)

                    @pl.when(c + NBS < CHUNKS)
                    def _():
                        nxt = c + NBS
                        pltpu.make_async_copy(rbuf[b], acc.at[dbuf[b]],
                                              ssem[b]).wait()
                        pltpu.async_copy(
                            dst_hbm.at[pl.ds(base + nxt * K, K)], dbuf[b],
                            isem[b])
                        pltpu.async_copy(
                            hs_hbm.at[sidx_v.at[pl.ds(nxt * K, K)]], rbuf[b],
                            gsem[b])

        for b in range(NBS):
            pltpu.make_async_copy(rbuf[b], acc.at[dbuf[b]], ssem[b]).wait()

        plsc.subcore_barrier()

        @pl.loop(0, NT)
        def _(t):
            u = t * NS + sid

            @pl.when(u < NU)
            def _():
                pltpu.async_copy(acc.at[pl.ds(u * K, K)],
                                 out_hbm.at[pl.ds(cid * N + u * K, K)],
                                 gsem[0])

        @pl.loop(0, NT)
        def _(t):
            u = t * NS + sid

            @pl.when(u < NU)
            def _():
                pltpu.make_async_copy(acc.at[pl.ds(u * K, K)],
                                      out_hbm.at[pl.ds(cid * N + u * K, K)],
                                      gsem[0]).wait()

    return scatter_kernel


_deg_call = _make_deg_kernel()
_scatter128 = _make_scatter_kernel(D)


def _dinv_of(degp):
    return lax.rsqrt(degp[0] + degp[1] + 1.0)


def _mm1_body(degp_ref, x_ref, w_ref, o_ref):
    dinv = _dinv_of(degp_ref[...])
    h = jnp.dot(x_ref[...], w_ref[...],
                preferred_element_type=_f32,
                precision=lax.Precision.HIGHEST)
    o_ref[...] = h * dinv[:, None]


def _layer2_body(degp_ref, agg_ref, hs1_ref, b1_ref, o_ref):
    dinv = _dinv_of(degp_ref[...])
    a = agg_ref[:N] + agg_ref[N:] + hs1_ref[...]
    out1 = jnp.maximum(dinv[:, None] * a + b1_ref[...][None, :], 0.0)
    o_ref[...] = out1 * dinv[:, None]


def _final_body(degp_ref, agg_ref, os1_ref, b2_ref, w2_ref, o_ref):
    dinv = _dinv_of(degp_ref[...])
    a = agg_ref[:N] + agg_ref[N:] + os1_ref[...]
    t = jnp.dot(a, w2_ref[...],
                preferred_element_type=_f32,
                precision=lax.Precision.HIGHEST)
    z = dinv[:, None] * t + b2_ref[...][None, :]
    z = jnp.maximum(z, 0.0)
    m = jnp.max(z, axis=1, keepdims=True)
    e = z - m
    lse = jnp.log(jnp.sum(jnp.exp(e), axis=1, keepdims=True))
    o_ref[...] = e - lse


def _tc(body, out_shape):
    return pl.pallas_call(body, out_shape=jax.ShapeDtypeStruct(out_shape, _f32))


def kernel(x, edge_index, W1, b1, W2, b2):
    src = edge_index[0].astype(jnp.int32)
    dst = edge_index[1].astype(jnp.int32)

    degp = _deg_call(dst).reshape(NC, N)

    hs1 = _tc(_mm1_body, (N, D))(degp, x, W1)

    agg1 = _scatter128(hs1, src, dst)

    os1 = _tc(_layer2_body, (N, D))(degp, agg1, hs1, b1)

    agg2 = _scatter128(os1, src, dst)

    return _tc(_final_body, (N, C))(degp, agg2, os1, b2, W2)


# speedup vs baseline: 37.5467x; 1.0223x over previous
import functools

import jax
import jax.numpy as jnp
from jax import lax
from jax.experimental import pallas as pl
from jax.experimental.pallas import tpu as pltpu
from jax.experimental.pallas import tpu_sc as plsc

N = 10000
E = 320000
D = 128
H = 128
C = 40
CP = 48

NC = 2
NS = 16
K = 80
EDGES_PER_SUB = E // (NC * NS)
CHUNKS = EDGES_PER_SUB // K
NB = 8
NBS = 3

_f32 = jnp.float32


def _mesh():
    return plsc.VectorSubcoreMesh(core_axis_name="c", subcore_axis_name="s")


def _make_deg_kernel():
    NU = N // K
    NT = -(-NU // NS)

    @functools.partial(
        pl.kernel,
        out_type=jax.ShapeDtypeStruct((NC * N,), _f32),
        mesh=_mesh(),
        scratch_types=(
            [pltpu.VMEM((K,), jnp.int32) for _ in range(NB)]
            + [pltpu.VMEM((K,), _f32),
               pltpu.VMEM((K,), _f32),
               pltpu.VMEM_SHARED((N,), _f32)]
            + [pltpu.SemaphoreType.DMA for _ in range(2 * NB)]
        ),
    )
    def deg_kernel(dst_hbm, out_hbm, *refs):
        dbuf = refs[0:NB]
        ones, vbuf, table = refs[NB:NB + 3]
        isem = refs[NB + 3:2 * NB + 3]
        ssem = refs[2 * NB + 3:3 * NB + 3]
        cid = lax.axis_index("c")
        sid = lax.axis_index("s")
        base = (cid * NS + sid) * EDGES_PER_SUB
        for j in range(K // 16):
            ones[pl.ds(16 * j, 16)] = jnp.full((16,), 1.0, _f32)
            vbuf[pl.ds(16 * j, 16)] = jnp.zeros((16,), _f32)

        for b in range(NB):
            pltpu.async_copy(dst_hbm.at[pl.ds(base + b * K, K)], dbuf[b],
                             isem[b])

        @pl.loop(0, NT)
        def _(t):
            u = t * NS + sid

            @pl.when(u < NU)
            def _():
                pltpu.async_copy(vbuf, table.at[pl.ds(u * K, K)], ssem[0])

        @pl.loop(0, NT)
        def _(t):
            u = t * NS + sid

            @pl.when(u < NU)
            def _():
                pltpu.make_async_copy(vbuf, table.at[pl.ds(u * K, K)],
                                      ssem[0]).wait()

        plsc.subcore_barrier()

        @pl.loop(0, CHUNKS, step=NB)
        def _(i):
            for b in range(NB):
                c = i + b

                @pl.when(c < CHUNKS)
                def _():
                    pltpu.make_async_copy(dst_hbm.at[pl.ds(base, K)],
                                          dbuf[b], isem[b]).wait()
                    pltpu.async_copy(ones, table.at[dbuf[b]], ssem[b],
                                     add=True)

                    @pl.when(c + NB < CHUNKS)
                    def _():
                        pltpu.make_async_copy(ones, table.at[dbuf[b]],
                                              ssem[b]).wait()
                        pltpu.async_copy(
                            dst_hbm.at[pl.ds(base + (c + NB) * K, K)],
                            dbuf[b], isem[b])

        for b in range(NB):
            pltpu.make_async_copy(ones, table.at[dbuf[b]], ssem[b]).wait()

        plsc.subcore_barrier()

        @pl.loop(0, NT)
        def _(t):
            u = t * NS + sid

            @pl.when(u < NU)
            def _():
                pltpu.sync_copy(table.at[pl.ds(u * K, K)], vbuf)
                pltpu.sync_copy(vbuf, out_hbm.at[pl.ds(cid * N + u * K, K)])

    return deg_kernel


def _make_scatter_kernel(F):
    NU = N // K
    NT = -(-NU // NS)

    @functools.partial(
        pl.kernel,
        out_type=jax.ShapeDtypeStruct((NC * N, F), _f32),
        mesh=_mesh(),
        scratch_types=(
            [pltpu.VMEM((EDGES_PER_SUB,), jnp.int32)]
            + [pltpu.VMEM((K,), jnp.int32) for _ in range(NBS)]
            + [pltpu.VMEM((K, F), _f32) for _ in range(NBS)]
            + [pltpu.VMEM_SHARED((N, F), _f32)]
            + [pltpu.SemaphoreType.DMA for _ in range(3 * NBS)]
        ),
    )
    def scatter_kernel(hs_hbm, src_hbm, dst_hbm, out_hbm, *refs):
        sidx_v = refs[0]
        dbuf = refs[1:1 + NBS]
        rbuf = refs[1 + NBS:1 + 2 * NBS]
        acc = refs[1 + 2 * NBS]
        isem = refs[2 + 2 * NBS:2 + 3 * NBS]
        gsem = refs[2 + 3 * NBS:2 + 4 * NBS]
        ssem = refs[2 + 4 * NBS:2 + 5 * NBS]
        cid = lax.axis_index("c")
        sid = lax.axis_index("s")
        base = (cid * NS + sid) * EDGES_PER_SUB

        pltpu.sync_copy(src_hbm.at[pl.ds(base, EDGES_PER_SUB)], sidx_v)

        @pl.loop(0, K)
        def _(r):
            for j in range(F // 16):
                rbuf[0][r, pl.ds(16 * j, 16)] = jnp.zeros((16,), _f32)

        @pl.loop(0, NT)
        def _(t):
            u = t * NS + sid

            @pl.when(u < NU)
            def _():
                pltpu.async_copy(rbuf[0], acc.at[pl.ds(u * K, K)], gsem[0])

        @pl.loop(0, NT)
        def _(t):
            u = t * NS + sid

            @pl.when(u < NU)
            def _():
                pltpu.make_async_copy(rbuf[0], acc.at[pl.ds(u * K, K)],
                                      gsem[0]).wait()

        for b in range(NBS):
            pltpu.async_copy(dst_hbm.at[pl.ds(base + b * K, K)], dbuf[b],
                             isem[b])
            pltpu.async_copy(hs_hbm.at[sidx_v.at[pl.ds(b * K, K)]], rbuf[b],
                             gsem[b])

        plsc.subcore_barrier()

        @pl.loop(0, CHUNKS, step=NBS)
        def _(i):
            for b in range(NBS):
                c = i + b

                @pl.when(c < CHUNKS)
                def _():
                    pltpu.make_async_copy(dst_hbm.at[pl.ds(base, K)],
                                          dbuf[b], isem[b]).wait()
                    pltpu.make_async_copy(
                        hs_hbm.at[sidx_v.at[pl.ds(0, K)]], rbuf[b],
                        gsem[b]).wait()
                    pltpu.async_copy(rbuf[b], acc.at[dbuf[b]], ssem[b],
                                     add=True)

                    @pl.when(c + NBS < CHUNKS)
                    def _():
                        nxt = c + NBS
                        pltpu.make_async_copy(rbuf[b], acc.at[dbuf[b]],
                                              ssem[b]).wait()
                        pltpu.async_copy(
                            dst_hbm.at[pl.ds(base + nxt * K, K)], dbuf[b],
                            isem[b])
                        pltpu.async_copy(
                            hs_hbm.at[sidx_v.at[pl.ds(nxt * K, K)]], rbuf[b],
                            gsem[b])

        for b in range(NBS):
            pltpu.make_async_copy(rbuf[b], acc.at[dbuf[b]], ssem[b]).wait()

        plsc.subcore_barrier()

        @pl.loop(0, NT)
        def _(t):
            u = t * NS + sid

            @pl.when(u < NU)
            def _():
                pltpu.async_copy(acc.at[pl.ds(u * K, K)],
                                 out_hbm.at[pl.ds(cid * N + u * K, K)],
                                 gsem[0])

        @pl.loop(0, NT)
        def _(t):
            u = t * NS + sid

            @pl.when(u < NU)
            def _():
                pltpu.make_async_copy(acc.at[pl.ds(u * K, K)],
                                      out_hbm.at[pl.ds(cid * N + u * K, K)],
                                      gsem[0]).wait()

    return scatter_kernel


_deg_call = _make_deg_kernel()
_scatter128 = _make_scatter_kernel(D)


def _dinv_of(degp):
    return lax.rsqrt(degp[0] + degp[1] + 1.0)


def _mm1_body(degp_ref, x_ref, w_ref, o_ref):
    dinv = _dinv_of(degp_ref[...])
    h = jnp.dot(x_ref[...], w_ref[...],
                preferred_element_type=_f32,
                precision=lax.Precision.HIGHEST)
    o_ref[...] = h * dinv[:, None]


def _layer2_body(degp_ref, agg_ref, hs1_ref, b1_ref, o_ref):
    dinv = _dinv_of(degp_ref[...])
    a = agg_ref[:N] + agg_ref[N:] + hs1_ref[...]
    out1 = jnp.maximum(dinv[:, None] * a + b1_ref[...][None, :], 0.0)
    o_ref[...] = out1 * dinv[:, None]


def _final_body(degp_ref, agg_ref, os1_ref, b2_ref, w2_ref, o_ref):
    dinv = _dinv_of(degp_ref[...])
    a = agg_ref[:N] + agg_ref[N:] + os1_ref[...]
    t = jnp.dot(a, w2_ref[...],
                preferred_element_type=_f32,
                precision=lax.Precision.HIGHEST)
    z = dinv[:, None] * t + b2_ref[...][None, :]
    z = jnp.maximum(z, 0.0)
    m = jnp.max(z, axis=1, keepdims=True)
    e = z - m
    lse = jnp.log(jnp.sum(jnp.exp(e), axis=1, keepdims=True))
    o_ref[...] = e - lse


def _tc(body, out_shape):
    return pl.pallas_call(body, out_shape=jax.ShapeDtypeStruct(out_shape, _f32))


def kernel(x, edge_index, W1, b1, W2, b2):
    src = edge_index[0].astype(jnp.int32)
    dst = edge_index[1].astype(jnp.int32)

    degp = _deg_call(dst).reshape(NC, N)

    hs1 = _tc(_mm1_body, (N, D))(degp, x, W1)

    agg1 = _scatter128(hs1, src, dst)

    os1 = _tc(_layer2_body, (N, D))(degp, agg1, hs1, b1)

    agg2 = _scatter128(os1, src, dst)

    return _tc(_final_body, (N, C))(degp, agg2, os1, b2, W2)
